# baseline (speedup 1.0000x reference)
import functools

import jax
import jax.numpy as jnp
from jax import lax
from jax.experimental import pallas as pl
from jax.experimental.pallas import tpu as pltpu
from jax.experimental.pallas import tpu_sc as plsc

VOCAB = 1000000
EMBED_DIM = 32
HIDDEN_DIM = 64
NUM_CLASSES = 3
BATCH = 16384
HIST = 50

NC = 2
NS = 16
NW = NC * NS
B_PER_W = BATCH // NW
GROUP = 8
ROWS_PER_GROUP = GROUP * HIST
NGROUPS = B_PER_W // GROUP
TOKENS_PER_W = B_PER_W * HIST
NBUF = 4


def _sc_gather_pool(ids1d, emb):
    mesh = plsc.VectorSubcoreMesh(core_axis_name="c", subcore_axis_name="s",
                                  num_cores=NC, num_subcores=NS)

    @functools.partial(
        pl.kernel,
        out_type=jax.ShapeDtypeStruct((BATCH, EMBED_DIM), jnp.float32),
        mesh=mesh,
        scratch_types=[
            pltpu.VMEM((TOKENS_PER_W,), jnp.int32),
            pltpu.VMEM((NBUF, ROWS_PER_GROUP, EMBED_DIM), jnp.float32),
            pltpu.VMEM((NBUF, GROUP, EMBED_DIM), jnp.float32),
            pltpu.SemaphoreType.DMA((NBUF,)),
            pltpu.SemaphoreType.DMA((NBUF,)),
        ],
        compiler_params=pltpu.CompilerParams(use_tc_tiling_on_sc=False),
    )
    def k(ids_hbm, emb_hbm, out_hbm, ids_all, rows_v, out_v, gsem, osem):
        wid = lax.axis_index("s") * NC + lax.axis_index("c")
        out_base = wid * B_PER_W

        pltpu.sync_copy(ids_hbm.at[pl.ds(wid * TOKENS_PER_W, TOKENS_PER_W)],
                        ids_all)

        def fire_gather(s, g):
            pltpu.async_copy(
                emb_hbm.at[ids_all.at[pl.ds(g * ROWS_PER_GROUP,
                                            ROWS_PER_GROUP)]],
                rows_v.at[s], gsem.at[s])

        def drain_gather(s):
            pltpu.make_async_copy(
                emb_hbm.at[ids_all.at[pl.ds(0, ROWS_PER_GROUP)]],
                rows_v.at[s], gsem.at[s]).wait()

        def drain_out(s):
            pltpu.make_async_copy(out_v.at[s],
                                  out_hbm.at[pl.ds(0, GROUP)],
                                  osem.at[s]).wait()

        for s in range(NBUF - 1):
            fire_gather(s, s)

        def blk_body(j, carry):
            for s in range(NBUF):
                g = NBUF * j + s
                drain_gather(s)

                @pl.when(j > 0)
                def _():
                    drain_out(s)

                for b in range(GROUP):
                    base = b * HIST
                    for h in (0, 16):
                        acc = (rows_v[s, base, pl.ds(h, 16)]
                               + rows_v[s, base + HIST - 1, pl.ds(h, 16)])
                        for t in range(1, HIST - 1, 2):
                            pair = (rows_v[s, base + t, pl.ds(h, 16)]
                                    + rows_v[s, base + t + 1, pl.ds(h, 16)])
                            acc = acc + pair
                        out_v[s, b, pl.ds(h, 16)] = acc
                pltpu.async_copy(out_v.at[s],
                                 out_hbm.at[pl.ds(out_base + g * GROUP,
                                                  GROUP)],
                                 osem.at[s])

                @pl.when(g + NBUF - 1 < NGROUPS)
                def _():
                    fire_gather((g + NBUF - 1) % NBUF, g + NBUF - 1)
            return carry

        lax.fori_loop(0, NGROUPS // NBUF, blk_body, 0)
        for s in range(NBUF):
            drain_out(s)

    return k(ids1d, emb)


NBLK_FULL = VOCAB // 128
TAIL_BASE = NBLK_FULL * 128
TBUF = 4


def _sc_transpose(embT, tail_pack):
    mesh = plsc.VectorSubcoreMesh(core_axis_name="c", subcore_axis_name="s",
                                  num_cores=NC, num_subcores=NS)

    @functools.partial(
        pl.kernel,
        out_type=jax.ShapeDtypeStruct((VOCAB // 4, 128), jnp.float32),
        mesh=mesh,
        scratch_types=[
            pltpu.VMEM((TBUF, EMBED_DIM, 129), jnp.float32),
            pltpu.VMEM((TBUF, 32, 128), jnp.float32),
            pltpu.VMEM((16, 128), jnp.float32),
            pltpu.SemaphoreType.DMA((TBUF,)),
            pltpu.SemaphoreType.DMA((TBUF,)),
        ],
        compiler_params=pltpu.CompilerParams(use_tc_tiling_on_sc=True,
                                             needs_layout_passes=False),
    )
    def k(embT_hbm, tail_hbm, out_hbm, in_buf, out_buf, tail_buf,
          isem, osem):
        wid = lax.axis_index("s") * NC + lax.axis_index("c")
        iota16 = lax.iota(jnp.int32, 16)

        def fire_in(s, kk):
            blk = kk * NW + wid
            pltpu.async_copy(
                embT_hbm.at[pl.ds(0, EMBED_DIM), pl.ds(blk * 128, 128)],
                in_buf.at[s].at[pl.ds(0, EMBED_DIM), pl.ds(0, 128)],
                isem.at[s])

        def drain_in(s):
            pltpu.make_async_copy(
                embT_hbm.at[pl.ds(0, EMBED_DIM), pl.ds(0, 128)],
                in_buf.at[s].at[pl.ds(0, EMBED_DIM), pl.ds(0, 128)],
                isem.at[s]).wait()

        def drain_out(s):
            pltpu.make_async_copy(out_buf.at[s], out_hbm.at[pl.ds(0, 32)],
                                  osem.at[s]).wait()

        def compute(s):
            for p in range(32):
                for a in range(4):
                    col = jnp.full((16,), 4 * p + a, jnp.int32)
                    for h in (0, 1):
                        v = plsc.load_gather(in_buf.at[s],
                                             [iota16 + 16 * h, col])
                        out_buf[s, p, pl.ds(32 * a + 16 * h, 16)] = v

        NK_MAIN = NBLK_FULL // NW
        for s in range(TBUF - 1):
            fire_in(s, s)

        def blk_body(j, carry):
            for s in range(TBUF):
                kk = TBUF * j + s
                blk = kk * NW + wid
                drain_in(s)

                @pl.when(j > 0)
                def _():
                    drain_out(s)

                compute(s)
                pltpu.async_copy(out_buf.at[s],
                                 out_hbm.at[pl.ds(blk * 32, 32)],
                                 osem.at[s])

                @pl.when(kk + TBUF - 1 < NK_MAIN)
                def _():
                    fire_in((kk + TBUF - 1) % TBUF, kk + TBUF - 1)
            return carry

        lax.fori_loop(0, NK_MAIN // TBUF, blk_body, 0)
        for s in range(TBUF):
            drain_out(s)

        @pl.when(wid < NBLK_FULL - NK_MAIN * NW)
        def _():
            fire_in(0, NK_MAIN)
            drain_in(0)
            compute(0)
            pltpu.async_copy(
                out_buf.at[0],
                out_hbm.at[pl.ds((NK_MAIN * NW + wid) * 32, 32)],
                osem.at[0])
            drain_out(0)

        @pl.when(wid == 4)
        def _():
            pltpu.sync_copy(tail_hbm, tail_buf)
            pltpu.sync_copy(tail_buf,
                            out_hbm.at[pl.ds(NBLK_FULL * 32, 16)])

    return k(embT, tail_pack)


def _tc_mlp(pooled, W1, b1, W2, b2):
    tile = 2048
    scale = 1.0 / HIST

    def body(x_ref, w1_ref, b1_ref, w2_ref, b2_ref, o_ref):
        x = x_ref[...] * scale
        h = jnp.dot(x, w1_ref[...], preferred_element_type=jnp.float32)
        h = jnp.maximum(h + b1_ref[...], 0.0)
        o_ref[...] = (jnp.dot(h, w2_ref[...],
                              preferred_element_type=jnp.float32)
                      + b2_ref[...])

    return pl.pallas_call(
        body,
        grid=(BATCH // tile,),
        in_specs=[
            pl.BlockSpec((tile, EMBED_DIM), lambda i: (i, 0)),
            pl.BlockSpec((EMBED_DIM, HIDDEN_DIM), lambda i: (0, 0)),
            pl.BlockSpec((1, HIDDEN_DIM), lambda i: (0, 0)),
            pl.BlockSpec((HIDDEN_DIM, NUM_CLASSES), lambda i: (0, 0)),
            pl.BlockSpec((1, NUM_CLASSES), lambda i: (0, 0)),
        ],
        out_specs=pl.BlockSpec((tile, NUM_CLASSES), lambda i: (i, 0)),
        out_shape=jax.ShapeDtypeStruct((BATCH, NUM_CLASSES), jnp.float32),
    )(pooled, W1, b1.reshape(1, HIDDEN_DIM), W2, b2.reshape(1, NUM_CLASSES))


def kernel(ids, emb, W1, b1, W2, b2):
    ids1d = ids.astype(jnp.int32).reshape(-1)
    tail_pack = emb[TAIL_BASE:].reshape(16, 128)
    emb_rm = _sc_transpose(emb.T, tail_pack).reshape(VOCAB, EMBED_DIM)
    pooled = _sc_gather_pool(ids1d, emb_rm)
    return _tc_mlp(pooled, W1, b1, W2, b2)


# speedup vs baseline: 1.7516x; 1.7516x over previous
import functools

import jax
import jax.numpy as jnp
from jax import lax
from jax.experimental import pallas as pl
from jax.experimental.pallas import tpu as pltpu
from jax.experimental.pallas import tpu_sc as plsc

VOCAB = 1000000
EMBED_DIM = 32
HIDDEN_DIM = 64
NUM_CLASSES = 3
BATCH = 16384
HIST = 50

NC = 2
NS = 16
NW = NC * NS
B_PER_W = BATCH // NW
GROUP = 8
IDS_PER_ROW = 100
ROWS_PER_GROUP = GROUP * HIST
IDROWS_PER_GROUP = ROWS_PER_GROUP // IDS_PER_ROW
NGROUPS = B_PER_W // GROUP


def _sc_gather_pool(ids2d, emb):
    mesh = plsc.VectorSubcoreMesh(core_axis_name="c", subcore_axis_name="s",
                                  num_cores=NC, num_subcores=NS)
    idrows_per_w = NGROUPS * IDROWS_PER_GROUP

    @functools.partial(
        pl.kernel,
        out_type=jax.ShapeDtypeStruct((BATCH, EMBED_DIM), jnp.float32),
        mesh=mesh,
        scratch_types=[
            pltpu.VMEM((idrows_per_w, IDS_PER_ROW), jnp.int32),
            pltpu.VMEM((2, ROWS_PER_GROUP, EMBED_DIM), jnp.float32),
            pltpu.VMEM((2, GROUP, EMBED_DIM), jnp.float32),
            pltpu.SemaphoreType.DMA((2,)),
            pltpu.SemaphoreType.DMA((2,)),
        ],
        compiler_params=pltpu.CompilerParams(use_tc_tiling_on_sc=False),
    )
    def k(ids_hbm, emb_hbm, out_hbm, ids_all, rows_v, out_v, gsem, osem):
        wid = lax.axis_index("s") * NC + lax.axis_index("c")
        out_base = wid * B_PER_W

        pltpu.sync_copy(ids_hbm.at[pl.ds(wid * idrows_per_w, idrows_per_w)],
                        ids_all)

        def fire_gathers(s, g):
            for j in range(IDROWS_PER_GROUP):
                pltpu.async_copy(
                    emb_hbm.at[ids_all.at[g * IDROWS_PER_GROUP + j]],
                    rows_v.at[s].at[pl.ds(j * IDS_PER_ROW, IDS_PER_ROW)],
                    gsem.at[s])

        def drain_gathers(s):
            for j in range(IDROWS_PER_GROUP):
                pltpu.make_async_copy(
                    emb_hbm.at[ids_all.at[0]],
                    rows_v.at[s].at[pl.ds(j * IDS_PER_ROW, IDS_PER_ROW)],
                    gsem.at[s]).wait()

        def drain_out(s):
            pltpu.make_async_copy(out_v.at[s],
                                  out_hbm.at[pl.ds(0, GROUP)],
                                  osem.at[s]).wait()

        fire_gathers(0, 0)
        fire_gathers(1, 1)

        def pair_body(i, carry):
            for s in (0, 1):
                g = 2 * i + s
                drain_gathers(s)

                @pl.when(i > 0)
                def _():
                    drain_out(s)

                for b in range(GROUP):
                    base = b * HIST
                    for h in (0, 16):
                        acc = (rows_v[s, base, pl.ds(h, 16)]
                               + rows_v[s, base + HIST - 1, pl.ds(h, 16)])
                        for t in range(1, HIST - 1, 2):
                            pair = (rows_v[s, base + t, pl.ds(h, 16)]
                                    + rows_v[s, base + t + 1, pl.ds(h, 16)])
                            acc = acc + pair
                        out_v[s, b, pl.ds(h, 16)] = acc
                pltpu.async_copy(out_v.at[s],
                                 out_hbm.at[pl.ds(out_base + g * GROUP,
                                                  GROUP)],
                                 osem.at[s])

                @pl.when(g + 2 < NGROUPS)
                def _():
                    fire_gathers(s, g + 2)
            return carry

        lax.fori_loop(0, NGROUPS // 2, pair_body, 0)
        drain_out(0)
        drain_out(1)

    return k(ids2d, emb)


def _tc_mlp(pooled, W1, b1, W2, b2):
    tile = 2048
    scale = 1.0 / HIST

    def body(x_ref, w1_ref, b1_ref, w2_ref, b2_ref, o_ref):
        x = x_ref[...] * scale
        h = jnp.dot(x, w1_ref[...], preferred_element_type=jnp.float32)
        h = jnp.maximum(h + b1_ref[...], 0.0)
        o_ref[...] = (jnp.dot(h, w2_ref[...],
                              preferred_element_type=jnp.float32)
                      + b2_ref[...])

    return pl.pallas_call(
        body,
        grid=(BATCH // tile,),
        in_specs=[
            pl.BlockSpec((tile, EMBED_DIM), lambda i: (i, 0)),
            pl.BlockSpec((EMBED_DIM, HIDDEN_DIM), lambda i: (0, 0)),
            pl.BlockSpec((1, HIDDEN_DIM), lambda i: (0, 0)),
            pl.BlockSpec((HIDDEN_DIM, NUM_CLASSES), lambda i: (0, 0)),
            pl.BlockSpec((1, NUM_CLASSES), lambda i: (0, 0)),
        ],
        out_specs=pl.BlockSpec((tile, NUM_CLASSES), lambda i: (i, 0)),
        out_shape=jax.ShapeDtypeStruct((BATCH, NUM_CLASSES), jnp.float32),
    )(pooled, W1, b1.reshape(1, HIDDEN_DIM), W2, b2.reshape(1, NUM_CLASSES))


def kernel(ids, emb, W1, b1, W2, b2):
    ids2d = ids.astype(jnp.int32).reshape(BATCH * HIST // IDS_PER_ROW,
                                          IDS_PER_ROW)
    pooled = _sc_gather_pool(ids2d, emb)
    return _tc_mlp(pooled, W1, b1, W2, b2)
